# baseline (speedup 1.0000x reference)
import functools

import jax
import jax.numpy as jnp
from jax import lax
from jax.experimental import pallas as pl
from jax.experimental.pallas import tpu as pltpu
from jax.experimental.pallas import tpu_sc as plsc

_G = 4096
_GP = 4224
_ZROWS = _GP // 16
_ZCHUNK = _ZROWS // 11
_SCAP = 16
_W = 128


def _zeros16():
    return jnp.zeros((16,), jnp.float32)


def _sink16():
    return jnp.full((16,), _G, jnp.int32)


def _sc_pool_body(*refs, NR, BR, D, KMAX, CNT, CBR):
    if CBR:
        (data_hbm, ids_hbm, cids_hbm, out_hbm, bufA, bufB, idbufA, idbufB,
         zbuf, stage_v, stage_iv, acc_buf, table, semA, semB) = refs
    else:
        (data_hbm, ids_hbm, out_hbm, bufA, bufB, idbufA, idbufB,
         zbuf, stage_v, stage_iv, acc_buf, table, semA, semB) = refs
    cid = lax.axis_index("c")
    sid = lax.axis_index("s")
    tid = cid * 16 + sid
    nvec = D // 16

    def zb(i, _):
        for j in range(_W // 16):
            zbuf[i, pl.ds(16 * j, 16)] = _zeros16()
        return 0
    lax.fori_loop(0, _ZCHUNK, zb, 0)
    for i in range(11):
        pltpu.sync_copy(zbuf,
                        table.at[pl.ds(sid * _ZROWS + i * _ZCHUNK, _ZCHUNK)])
    for s in range(_SCAP):
        for j in range(_W // 16):
            stage_v[s, pl.ds(16 * j, 16)] = _zeros16()
    stage_iv[0, pl.ds(0, 16)] = _sink16()
    plsc.subcore_barrier()

    iota16 = lax.iota(jnp.int32, 16)

    def push(seg, sn, write):
        write(sn)
        iv = stage_iv[0, pl.ds(0, 16)]
        stage_iv[0, pl.ds(0, 16)] = jnp.where(iota16 == sn, seg, iv)
        sn2 = sn + 1

        def dodma(_):
            pltpu.sync_copy(stage_v, table.at[stage_iv.at[0]], add=True)
            stage_iv[0, pl.ds(0, 16)] = _sink16()
            return jnp.int32(0)
        return lax.cond(sn2 == _SCAP, dodma, lambda _: sn2, 0)

    def flush(seg, cnt, sn):
        def do(sn):
            def write(sn):
                for j in range(nvec):
                    stage_v[sn, pl.ds(16 * j, 16)] = acc_buf[0, pl.ds(16 * j, 16)]
                if CNT:
                    stage_v[sn, pl.ds(D, 16)] = jnp.where(
                        iota16 == 0, cnt.astype(jnp.float32), 0.0)
            return push(seg, sn, write)
        return lax.cond(cnt > 0, do, lambda s: s, sn)

    def cflush(seg, cnt, sn):
        def do(sn):
            def write(sn):
                stage_v[sn, pl.ds(D, 16)] = jnp.where(
                    iota16 == 1, cnt.astype(jnp.float32), 0.0)
            return push(seg, sn, write)
        return lax.cond(cnt > 0, do, lambda s: s, sn)

    def run_groups(idbuf, ngroups, sn, on_group, do_flush):
        def group_body(g, carry):
            cur, cnt, sn = carry
            i0 = g * 16
            idvec = idbuf[pl.ds(i0, 16)]
            a = idvec[0]
            z = idvec[15]

            def fast(cur, cnt, sn):
                adder = on_group(g)

                def new_seg(sn):
                    sn2 = do_flush(cur, cnt, sn)
                    adder(True)
                    return a, jnp.int32(16), sn2

                def same_seg(sn):
                    adder(False)
                    return cur, cnt + 16, sn
                return lax.cond(a != cur, new_seg, same_seg, sn)

            def slow(cur, cnt, sn):
                for k in range(16):
                    idv = idvec[k]
                    new = idv != cur
                    rowadd = on_group(g, k)

                    def donew(sn, cnt=cnt, cur=cur, rowadd=rowadd):
                        sn2 = do_flush(cur, cnt, sn)
                        rowadd(True)
                        return sn2

                    def doold(sn, rowadd=rowadd):
                        rowadd(False)
                        return sn
                    sn = lax.cond(new, donew, doold, sn)
                    cnt = jnp.where(new, 1, cnt + 1)
                    cur = idv
                return cur, cnt, sn

            return lax.cond(a == z, fast, slow, cur, cnt, sn)

        cur, cnt, sn = lax.fori_loop(
            0, ngroups, group_body, (jnp.int32(-1), jnp.int32(0), sn))
        return do_flush(cur, cnt, sn)

    def process(buf, idbuf, sn):
        def on_group(g, k=None):
            i0 = g * 16

            def vreg(k, j):
                return buf[i0 + k, pl.ds(16 * j, 16)]

            if k is None:
                sums = []
                for j in range(nvec):
                    rows = [vreg(k2, j) for k2 in range(16)]
                    while len(rows) > 1:
                        rows = ([rows[p] + rows[p + 1]
                                 for p in range(0, len(rows) - 1, 2)]
                                + (rows[-1:] if len(rows) % 2 else []))
                    sums.append(rows[0])
            else:
                sums = [vreg(k, j) for j in range(nvec)]

            def adder(reset):
                for j in range(nvec):
                    if reset:
                        acc_buf[0, pl.ds(16 * j, 16)] = sums[j]
                    else:
                        acc_buf[0, pl.ds(16 * j, 16)] = (
                            acc_buf[0, pl.ds(16 * j, 16)] + sums[j])
            return adder
        return run_groups(idbuf, BR // 16, sn, on_group, flush)

    sn0 = jnp.int32(0)
    if CBR:
        pltpu.sync_copy(cids_hbm.at[pl.ds(tid * CBR, CBR)],
                        idbufA.at[pl.ds(0, CBR)])
        sn0 = run_groups(idbufA, CBR // 16, sn0,
                         lambda g, k=None: (lambda reset: None), cflush)

    def issue(k, buf, idbuf, sem):
        b = tid + 32 * k
        r0 = jnp.minimum(b * BR, NR - BR)
        pltpu.async_copy(data_hbm.at[pl.ds(r0, BR)], buf, sem)
        pltpu.async_copy(ids_hbm.at[pl.ds(b * BR, BR)], idbuf, sem)

    def wait(buf, idbuf, sem):
        pltpu.make_async_copy(data_hbm.at[pl.ds(0, BR)], buf, sem).wait()
        pltpu.make_async_copy(ids_hbm.at[pl.ds(0, BR)], idbuf, sem).wait()

    issue(0, bufA, idbufA, semA)
    if KMAX > 1:
        issue(1, bufB, idbufB, semB)

    def pair_body(m, sn):
        k0 = 2 * m
        wait(bufA, idbufA, semA)
        sn = process(bufA, idbufA, sn)
        lax.cond(k0 + 2 < KMAX,
                 lambda: issue(k0 + 2, bufA, idbufA, semA) or 0,
                 lambda: 0)
        wait(bufB, idbufB, semB)
        sn = process(bufB, idbufB, sn)
        lax.cond(k0 + 3 < KMAX,
                 lambda: issue(k0 + 3, bufB, idbufB, semB) or 0,
                 lambda: 0)
        return sn

    sn = lax.fori_loop(0, KMAX // 2, pair_body, sn0)
    if KMAX % 2:
        wait(bufA, idbufA, semA)
        sn = process(bufA, idbufA, sn)

    pltpu.sync_copy(stage_v, table.at[stage_iv.at[0]], add=True)

    plsc.subcore_barrier()
    pltpu.sync_copy(table.at[pl.ds(sid * _ZROWS, _ZROWS)],
                    out_hbm.at[cid, pl.ds(sid * _ZROWS, _ZROWS)])


def _sc_pool(data, ids, block_rows, cids=None):
    n, d = data.shape
    br = block_rows
    nb = ids.shape[0] // br
    assert nb * br == ids.shape[0] and nb % 32 == 0 and br % 16 == 0
    assert nb * br >= n and n >= br
    kmax = nb // 32
    cbr = 0 if cids is None else cids.shape[0] // 32
    mesh = plsc.VectorSubcoreMesh(core_axis_name="c", subcore_axis_name="s")
    body = functools.partial(
        _sc_pool_body, NR=n, BR=br, D=d, KMAX=kmax,
        CNT=(d + 32 <= _W), CBR=cbr)
    args = (data, ids) if cids is None else (data, ids, cids)
    return pl.kernel(
        body,
        out_type=jax.ShapeDtypeStruct((2, _GP, _W), jnp.float32),
        mesh=mesh,
        compiler_params=pltpu.CompilerParams(use_tc_tiling_on_sc=False),
        scratch_types=[
            pltpu.VMEM((br, d), jnp.float32),
            pltpu.VMEM((br, d), jnp.float32),
            pltpu.VMEM((max(br, cbr),), jnp.int32),
            pltpu.VMEM((br,), jnp.int32),
            pltpu.VMEM((_ZCHUNK, _W), jnp.float32),
            pltpu.VMEM((_SCAP, _W), jnp.float32),
            pltpu.VMEM((1, 16), jnp.int32),
            pltpu.VMEM((1, d), jnp.float32),
            pltpu.VMEM_SHARED((_GP, _W), jnp.float32),
            pltpu.SemaphoreType.DMA,
            pltpu.SemaphoreType.DMA,
        ],
    )(*args)


def _mlp_body(pb_ref, ps_ref, st_ref,
              w1b_ref, w1s_ref, w1t_ref, b1_ref,
              w2_ref, b2_ref, w3_ref, b3_ref, out_ref):
    pb = pb_ref[0] + pb_ref[1]
    ps = ps_ref[0] + ps_ref[1]
    bp = pb[:_G, :16] / jnp.maximum(pb[:_G, 16:17], 1.0)
    sp = ps[:_G, :128] / jnp.maximum(pb[:_G, 17:18], 1.0)

    def dot(a, b):
        return jax.lax.dot_general(a, b, (((1,), (0,)), ((), ())),
                                   preferred_element_type=jnp.float32)

    h = dot(bp, w1b_ref[...]) + dot(sp, w1s_ref[...]) + dot(st_ref[...], w1t_ref[...])
    h = jnp.maximum(h + b1_ref[...], 0.0)
    h = jnp.maximum(dot(h, w2_ref[...]) + b2_ref[...], 0.0)
    out_ref[...] = jnp.maximum(dot(h, w3_ref[...]) + b3_ref[...], 0.0)


def kernel(sites, bonds, states, graph_to_sites, graph_to_bonds,
           W1, b1, W2, b2, W3, b3):
    num_graphs, state_len = states.shape
    bond_len = bonds.shape[1]
    site_len = sites.shape[1]
    assert num_graphs == _G and bond_len == 16 and site_len == 128

    n_sites = sites.shape[0]
    n_bonds = bonds.shape[0]

    cpad = (-n_sites) % (32 * 16)
    cids = jnp.pad(graph_to_sites, (0, cpad), constant_values=_G)
    pb = _sc_pool(bonds, graph_to_bonds, 2000, cids)

    sbr = 320
    snb = -(-n_sites // sbr)
    snb += (-snb) % 32
    p = jnp.arange(snb * sbr, dtype=jnp.int32)
    b = p // sbr
    g = jnp.minimum(b * sbr, n_sites - sbr) + p % sbr
    ids_s = jnp.where(g >= b * sbr, graph_to_sites[g], _G)
    ps = _sc_pool(sites, ids_s, sbr)

    w1 = W1.T
    w1b = w1[:bond_len]
    w1s = w1[bond_len:bond_len + site_len]
    w1t = w1[bond_len + site_len:]

    out = pl.pallas_call(
        _mlp_body,
        out_shape=jax.ShapeDtypeStruct((num_graphs, state_len), jnp.float32),
    )(pb, ps, states,
      w1b, w1s, w1t, b1.reshape(1, -1),
      W2.T, b2.reshape(1, -1), W3.T, b3.reshape(1, -1))
    return out
